# baseline (speedup 1.0000x reference)
import jax
import jax.numpy as jnp
from jax import lax
from jax.experimental import pallas as pl
from jax.experimental.pallas import tpu as pltpu

N = 1024
IN_DIM = 512
HID = 256
OUT = 128
LOG2E = 1.4426950408889634


def _leaky(v):
    return jnp.maximum(v, 0.25 * v)


def _encoder_body(x_ref, adj_ref, w1_ref, b1_ref, w2_ref, b2_ref, wg_ref,
                  a_ref, mu_ref, lv_ref):
    f32 = jnp.float32
    adj = adj_ref[...]
    s1 = jnp.dot(x_ref[...], w1_ref[...], preferred_element_type=f32)
    x1 = _leaky(jnp.dot(adj, s1, preferred_element_type=f32) + b1_ref[...])
    s2 = jnp.dot(x1, w2_ref[...], preferred_element_type=f32)
    x2 = _leaky(jnp.dot(adj, s2, preferred_element_type=f32) + b2_ref[...])
    h = jnp.dot(x2, wg_ref[...], preferred_element_type=f32)
    a1 = a_ref[:, :OUT]
    a2 = a_ref[:, OUT:]
    u = jnp.sum(h * a1, axis=1, keepdims=True) * LOG2E
    v = lax.dot_general(a2, h, (((1,), (1,)), ((), ())),
                        preferred_element_type=f32) * LOG2E
    shift = _leaky(u + jnp.max(v))
    w = jnp.exp2(_leaky(u + v) - shift)
    w = jnp.where(adj > 0, w, jnp.float32(0.0))
    acc = jnp.dot(w, h, preferred_element_type=f32)
    rowsum = jnp.sum(w, axis=1, keepdims=True)
    mean_h = jnp.sum(h, axis=0, keepdims=True) * (1.0 / N)
    acc = jnp.where(rowsum > 0, acc / rowsum, mean_h)
    out_t = _leaky(acc).T
    mu_ref[...] = out_t[: OUT // 2, :]
    lv_ref[...] = out_t[OUT // 2:, :]


def kernel(x, adj, W1, b1, W2, b2, Wg, a):
    mu_t, lv_t = pl.pallas_call(
        _encoder_body,
        out_shape=(jax.ShapeDtypeStruct((OUT // 2, N), jnp.float32),
                   jax.ShapeDtypeStruct((OUT // 2, N), jnp.float32)),
    )(x, adj, W1, b1.reshape(1, HID), W2, b2.reshape(1, HID), Wg,
      a.reshape(1, 2 * OUT))
    return mu_t.T, lv_t.T
